# Initial kernel scaffold; baseline (speedup 1.0000x reference)
import jax
import jax.numpy as jnp
from jax.experimental import pallas as pl


def kernel(q, r, k_emb, v_emb, Mk, Mv0, f_W, f_b, e_W, e_b, a_W, a_b, ab_W, ab_b, df_W, df_b):
    raise NotImplementedError("write your pallas kernel here")


# speedup vs baseline: 2.9186x; 2.9186x over previous
import jax
import jax.numpy as jnp
from jax.experimental import pallas as pl
from jax.experimental.pallas import tpu as pltpu

B, L = 128, 200
NUM_C, D, M = 10000, 128, 64
DH = 32
NDQ = D // DH
LC = 25
PP = 456


def _pre_kernel(q_sref, rf_ref, kemb_ref, catw_ref, const_ref, dfb_ref,
                w_ref, e_ref, a_ref, f2_ref, qd_ref, kbuf):
    l = pl.program_id(0)
    iota8 = jax.lax.broadcasted_iota(jnp.int32, (8, D), 0)
    for b in range(B):
        idx = q_sref[l, b]
        base = pl.multiple_of((idx >> 3) << 3, 8)
        chunk = kemb_ref[pl.ds(base, 8), :]
        row = jnp.sum(jnp.where(iota8 == (idx & 7), chunk, 0.0), axis=0,
                      keepdims=True)
        kbuf[pl.ds(b, 1), :] = row
    k = kbuf[...]
    g = jax.lax.dot_general(catw_ref[...], k, (((1,), (1,)), ((), ())),
                            preferred_element_type=jnp.float32)
    rf = rf_ref[0]
    e = jax.nn.sigmoid(g[0:D] + const_ref[0] + rf * const_ref[1])
    a = jnp.tanh(g[D:2 * D] + const_ref[2] + rf * const_ref[3])
    f2 = g[2 * D:3 * D]
    qd = jnp.tanh(g[3 * D:3 * D + 1] + dfb_ref[0])
    wl = g[3 * D + 1:3 * D + 1 + M]
    wmax = jnp.max(wl, axis=0, keepdims=True)
    wexp = jnp.exp(wl - wmax)
    wsm = wexp / jnp.sum(wexp, axis=0, keepdims=True)
    w_ref[0] = wsm
    e_ref[0] = e
    a_ref[0] = a
    f2_ref[0] = f2
    qd_ref[0] = qd


def _scan_kernel(w_ref, e_ref, a_ref, mv0_ref, rd_ref, mv):
    mv[...] = mv0_ref[...]

    def step(t, _):
        wt = w_ref[pl.ds(t * M, M)]
        et = e_ref[t]
        at = a_ref[t]
        cur = mv[...]
        rd_ref[t] = jnp.sum(wt * cur, axis=0)
        mv[...] = cur * (1.0 - wt * et[None]) + wt * at[None]
        return ()

    jax.lax.fori_loop(0, L, step, ())


def _post_kernel(rd_ref, f2_ref, qd_ref, fw1_ref, fb_ref, abw_ref, abb_ref,
                 p_ref):
    fw1 = fw1_ref[...]
    abw = abw_ref[...]
    for j in range(LC):
        rd = rd_ref[j]
        f = jnp.tanh(jnp.dot(fw1, rd, preferred_element_type=jnp.float32)
                     + f2_ref[j] + fb_ref[...])
        ab = jnp.tanh(jnp.dot(abw, f, preferred_element_type=jnp.float32)
                      + abb_ref[0])
        p_ref[j] = jax.nn.sigmoid(3.0 * ab - qd_ref[j])


def kernel(q, r, k_emb, v_emb, Mk, Mv0, f_W, f_b, e_W, e_b, a_W, a_b,
           ab_W, ab_b, df_W, df_b):
    f32 = jnp.float32
    qT = q.astype(jnp.int32).T
    rf = r.astype(f32).T.reshape(L, 1, B)
    fW1 = f_W[:, :D]
    fW2 = f_W[:, D:]
    catw = jnp.concatenate(
        [e_W, a_W, fW2, df_W, Mk,
         jnp.zeros((PP - (3 * D + 1 + M), D), f32)], axis=0)
    ev = e_W @ v_emb.T
    av = a_W @ v_emb.T
    const = jnp.stack([
        jnp.broadcast_to((ev[:, 0] + e_b)[:, None], (D, B)),
        jnp.broadcast_to((ev[:, 1] - ev[:, 0])[:, None], (D, B)),
        jnp.broadcast_to((av[:, 0] + a_b)[:, None], (D, B)),
        jnp.broadcast_to((av[:, 1] - av[:, 0])[:, None], (D, B)),
    ])
    dfb = jnp.broadcast_to(df_b.reshape(1, 1), (1, B)).reshape(1, 1, B)
    fb_bc = jnp.broadcast_to(f_b[:, None], (D, B))
    abb = jnp.broadcast_to(ab_b.reshape(1, 1), (1, B)).reshape(1, 1, B)
    mv0b = jnp.broadcast_to(Mv0[:, :, None], (M, D, B))

    w3, e3, a3, f23, qd3 = pl.pallas_call(
        _pre_kernel,
        grid_spec=pltpu.PrefetchScalarGridSpec(
            num_scalar_prefetch=1,
            grid=(L,),
            in_specs=[
                pl.BlockSpec((1, 1, B), lambda l, qs: (l, 0, 0)),
                pl.BlockSpec((NUM_C, D), lambda l, qs: (0, 0)),
                pl.BlockSpec((PP, D), lambda l, qs: (0, 0)),
                pl.BlockSpec((4, D, B), lambda l, qs: (0, 0, 0)),
                pl.BlockSpec((1, 1, B), lambda l, qs: (0, 0, 0)),
            ],
            out_specs=[
                pl.BlockSpec((1, M, B), lambda l, qs: (l, 0, 0)),
                pl.BlockSpec((1, D, B), lambda l, qs: (l, 0, 0)),
                pl.BlockSpec((1, D, B), lambda l, qs: (l, 0, 0)),
                pl.BlockSpec((1, D, B), lambda l, qs: (l, 0, 0)),
                pl.BlockSpec((1, 1, B), lambda l, qs: (l, 0, 0)),
            ],
            scratch_shapes=[pltpu.VMEM((B, D), f32)],
        ),
        out_shape=[
            jax.ShapeDtypeStruct((L, M, B), f32),
            jax.ShapeDtypeStruct((L, D, B), f32),
            jax.ShapeDtypeStruct((L, D, B), f32),
            jax.ShapeDtypeStruct((L, D, B), f32),
            jax.ShapeDtypeStruct((L, 1, B), f32),
        ],
        compiler_params=pltpu.CompilerParams(
            dimension_semantics=("parallel",),
            vmem_limit_bytes=48 * 1024 * 1024,
        ),
    )(qT, rf, k_emb, catw, const, dfb)

    wflat = w3.reshape(L * M, 1, B)
    reads = pl.pallas_call(
        _scan_kernel,
        grid=(NDQ,),
        in_specs=[
            pl.BlockSpec((L * M, 1, B), lambda i: (0, 0, 0)),
            pl.BlockSpec((L, DH, B), lambda i: (0, i, 0)),
            pl.BlockSpec((L, DH, B), lambda i: (0, i, 0)),
            pl.BlockSpec((M, DH, B), lambda i: (0, i, 0)),
        ],
        out_specs=pl.BlockSpec((L, DH, B), lambda i: (0, i, 0)),
        out_shape=jax.ShapeDtypeStruct((L, D, B), f32),
        scratch_shapes=[pltpu.VMEM((M, DH, B), f32)],
        compiler_params=pltpu.CompilerParams(
            dimension_semantics=("parallel",),
            vmem_limit_bytes=48 * 1024 * 1024,
        ),
    )(wflat, e3, a3, mv0b)

    p3 = pl.pallas_call(
        _post_kernel,
        grid=(L // LC,),
        in_specs=[
            pl.BlockSpec((LC, D, B), lambda i: (i, 0, 0)),
            pl.BlockSpec((LC, D, B), lambda i: (i, 0, 0)),
            pl.BlockSpec((LC, 1, B), lambda i: (i, 0, 0)),
            pl.BlockSpec((D, D), lambda i: (0, 0)),
            pl.BlockSpec((D, B), lambda i: (0, 0)),
            pl.BlockSpec((1, D), lambda i: (0, 0)),
            pl.BlockSpec((1, 1, B), lambda i: (0, 0, 0)),
        ],
        out_specs=pl.BlockSpec((LC, 1, B), lambda i: (i, 0, 0)),
        out_shape=jax.ShapeDtypeStruct((L, 1, B), f32),
        compiler_params=pltpu.CompilerParams(
            dimension_semantics=("parallel",),
            vmem_limit_bytes=48 * 1024 * 1024,
        ),
    )(reads, f23, qd3, fW1, fb_bc, ab_W, abb)

    return p3.reshape(L, B).T


# speedup vs baseline: 3.1830x; 1.0906x over previous
import jax
import jax.numpy as jnp
from jax.experimental import pallas as pl
from jax.experimental.pallas import tpu as pltpu

B, L = 128, 200
NUM_C, D, M = 10000, 128, 64
DH = 32
NDQ = D // DH
LC = 25
PP = 456


LCP = 4
MC = 16


def _pre_kernel(q_sref, rf_ref, kemb_ref, catw_ref, const_ref, dfb_ref,
                w_ref, e_ref, a_ref, f2_ref, qd_ref, kbuf):
    l0 = pl.program_id(0) * LCP
    iota8 = jax.lax.broadcasted_iota(jnp.int32, (8, D), 0)
    for j in range(LCP):
        for b in range(B):
            idx = q_sref[l0 + j, b]
            base = pl.multiple_of((idx >> 3) << 3, 8)
            chunk = kemb_ref[pl.ds(base, 8), :]
            row = jnp.sum(jnp.where(iota8 == (idx & 7), chunk, 0.0), axis=0,
                          keepdims=True)
            kbuf[pl.ds(j * B + b, 1), :] = row
    k = kbuf[...]
    g = jax.lax.dot_general(catw_ref[...], k, (((1,), (1,)), ((), ())),
                            preferred_element_type=jnp.float32)
    for j in range(LCP):
        gj = g[:, j * B:(j + 1) * B]
        rf = rf_ref[j]
        e = jax.nn.sigmoid(gj[0:D] + const_ref[0] + rf * const_ref[1])
        a = jnp.tanh(gj[D:2 * D] + const_ref[2] + rf * const_ref[3])
        qd = jnp.tanh(gj[3 * D:3 * D + 1] + dfb_ref[0])
        wl = gj[3 * D + 1:3 * D + 1 + M]
        wmax = jnp.max(wl, axis=0, keepdims=True)
        wexp = jnp.exp(wl - wmax)
        wsm = wexp / jnp.sum(wexp, axis=0, keepdims=True)
        w_ref[j] = wsm
        e_ref[j] = e
        a_ref[j] = a
        f2_ref[j] = gj[2 * D:3 * D]
        qd_ref[j] = qd


def _scan_kernel(w_ref, e_ref, a_ref, mv0_ref, rd_ref):
    for mc in range(M // MC):
        def step(t, rows):
            et = e_ref[t]
            at = a_ref[t]
            acc = None
            new_rows = []
            for j in range(MC):
                wt = w_ref[t * M + mc * MC + j]
                row = rows[j]
                z = wt * row
                acc = z if acc is None else acc + z
                new_rows.append(row - wt * (row * et - at))
            if mc == 0:
                rd_ref[t] = acc
            else:
                rd_ref[t] = rd_ref[t] + acc
            return tuple(new_rows)

        init = tuple(mv0_ref[mc * MC + j] for j in range(MC))
        jax.lax.fori_loop(0, L, step, init)


def _post_kernel(rd_ref, f2_ref, qd_ref, fw1_ref, fb_ref, abw_ref, abb_ref,
                 p_ref):
    fw1 = fw1_ref[...]
    abw = abw_ref[...]
    for j in range(LC):
        rd = rd_ref[j]
        f = jnp.tanh(jnp.dot(fw1, rd, preferred_element_type=jnp.float32)
                     + f2_ref[j] + fb_ref[...])
        ab = jnp.tanh(jnp.dot(abw, f, preferred_element_type=jnp.float32)
                      + abb_ref[0])
        p_ref[j] = jax.nn.sigmoid(3.0 * ab - qd_ref[j])


def kernel(q, r, k_emb, v_emb, Mk, Mv0, f_W, f_b, e_W, e_b, a_W, a_b,
           ab_W, ab_b, df_W, df_b):
    f32 = jnp.float32
    qT = q.astype(jnp.int32).T
    rf = r.astype(f32).T.reshape(L, 1, B)
    fW1 = f_W[:, :D]
    fW2 = f_W[:, D:]
    catw = jnp.concatenate(
        [e_W, a_W, fW2, df_W, Mk,
         jnp.zeros((PP - (3 * D + 1 + M), D), f32)], axis=0)
    ev = e_W @ v_emb.T
    av = a_W @ v_emb.T
    const = jnp.stack([
        jnp.broadcast_to((ev[:, 0] + e_b)[:, None], (D, B)),
        jnp.broadcast_to((ev[:, 1] - ev[:, 0])[:, None], (D, B)),
        jnp.broadcast_to((av[:, 0] + a_b)[:, None], (D, B)),
        jnp.broadcast_to((av[:, 1] - av[:, 0])[:, None], (D, B)),
    ])
    dfb = jnp.broadcast_to(df_b.reshape(1, 1), (1, B)).reshape(1, 1, B)
    fb_bc = jnp.broadcast_to(f_b[:, None], (D, B))
    abb = jnp.broadcast_to(ab_b.reshape(1, 1), (1, B)).reshape(1, 1, B)
    mv0b = jnp.broadcast_to(Mv0[:, :, None], (M, D, B))

    w3, e3, a3, f23, qd3 = pl.pallas_call(
        _pre_kernel,
        grid_spec=pltpu.PrefetchScalarGridSpec(
            num_scalar_prefetch=1,
            grid=(L // LCP,),
            in_specs=[
                pl.BlockSpec((LCP, 1, B), lambda l, qs: (l, 0, 0)),
                pl.BlockSpec((NUM_C, D), lambda l, qs: (0, 0)),
                pl.BlockSpec((PP, D), lambda l, qs: (0, 0)),
                pl.BlockSpec((4, D, B), lambda l, qs: (0, 0, 0)),
                pl.BlockSpec((1, 1, B), lambda l, qs: (0, 0, 0)),
            ],
            out_specs=[
                pl.BlockSpec((LCP, M, B), lambda l, qs: (l, 0, 0)),
                pl.BlockSpec((LCP, D, B), lambda l, qs: (l, 0, 0)),
                pl.BlockSpec((LCP, D, B), lambda l, qs: (l, 0, 0)),
                pl.BlockSpec((LCP, D, B), lambda l, qs: (l, 0, 0)),
                pl.BlockSpec((LCP, 1, B), lambda l, qs: (l, 0, 0)),
            ],
            scratch_shapes=[pltpu.VMEM((LCP * B, D), f32)],
        ),
        out_shape=[
            jax.ShapeDtypeStruct((L, M, B), f32),
            jax.ShapeDtypeStruct((L, D, B), f32),
            jax.ShapeDtypeStruct((L, D, B), f32),
            jax.ShapeDtypeStruct((L, D, B), f32),
            jax.ShapeDtypeStruct((L, 1, B), f32),
        ],
        compiler_params=pltpu.CompilerParams(
            dimension_semantics=("parallel",),
            vmem_limit_bytes=48 * 1024 * 1024,
        ),
    )(qT, rf, k_emb, catw, const, dfb)

    wflat = w3.reshape(L * M, 1, B)
    reads = pl.pallas_call(
        _scan_kernel,
        grid=(NDQ,),
        in_specs=[
            pl.BlockSpec((L * M, 1, B), lambda i: (0, 0, 0)),
            pl.BlockSpec((L, DH, B), lambda i: (0, i, 0)),
            pl.BlockSpec((L, DH, B), lambda i: (0, i, 0)),
            pl.BlockSpec((M, DH, B), lambda i: (0, i, 0)),
        ],
        out_specs=pl.BlockSpec((L, DH, B), lambda i: (0, i, 0)),
        out_shape=jax.ShapeDtypeStruct((L, D, B), f32),
        compiler_params=pltpu.CompilerParams(
            dimension_semantics=("parallel",),
            vmem_limit_bytes=48 * 1024 * 1024,
        ),
    )(wflat, e3, a3, mv0b)

    p3 = pl.pallas_call(
        _post_kernel,
        grid=(L // LC,),
        in_specs=[
            pl.BlockSpec((LC, D, B), lambda i: (i, 0, 0)),
            pl.BlockSpec((LC, D, B), lambda i: (i, 0, 0)),
            pl.BlockSpec((LC, 1, B), lambda i: (i, 0, 0)),
            pl.BlockSpec((D, D), lambda i: (0, 0)),
            pl.BlockSpec((D, B), lambda i: (0, 0)),
            pl.BlockSpec((1, D), lambda i: (0, 0)),
            pl.BlockSpec((1, 1, B), lambda i: (0, 0, 0)),
        ],
        out_specs=pl.BlockSpec((LC, 1, B), lambda i: (i, 0, 0)),
        out_shape=jax.ShapeDtypeStruct((L, 1, B), f32),
        compiler_params=pltpu.CompilerParams(
            dimension_semantics=("parallel",),
            vmem_limit_bytes=48 * 1024 * 1024,
        ),
    )(reads, f23, qd3, fW1, fb_bc, ab_W, abb)

    return p3.reshape(L, B).T


# speedup vs baseline: 3.7355x; 1.1736x over previous
import jax
import jax.numpy as jnp
from jax.experimental import pallas as pl
from jax.experimental.pallas import tpu as pltpu

B, L = 128, 200
NUM_C, D, M = 10000, 128, 64
DH = 32
NDQ = D // DH
LC = 25
PP = 456


LCP = 4
MC = 8


def _pre_kernel(q_sref, rf_ref, kemb_ref, catw_ref, const_ref, dfb_ref,
                w_ref, e_ref, a_ref, f2_ref, qd_ref, kbuf):
    l0 = pl.program_id(0) * LCP
    iota8 = jax.lax.broadcasted_iota(jnp.int32, (8, D), 0)
    for j in range(LCP):
        for b in range(B):
            idx = q_sref[l0 + j, b]
            base = pl.multiple_of((idx >> 3) << 3, 8)
            chunk = kemb_ref[pl.ds(base, 8), :]
            row = jnp.sum(jnp.where(iota8 == (idx & 7), chunk, 0.0), axis=0,
                          keepdims=True)
            kbuf[pl.ds(j * B + b, 1), :] = row
    k = kbuf[...]
    g = jax.lax.dot_general(catw_ref[...], k, (((1,), (1,)), ((), ())),
                            preferred_element_type=jnp.float32)
    for j in range(LCP):
        gj = g[:, j * B:(j + 1) * B]
        rf = rf_ref[j]
        e = jax.nn.sigmoid(gj[0:D] + const_ref[0] + rf * const_ref[1])
        a = jnp.tanh(gj[D:2 * D] + const_ref[2] + rf * const_ref[3])
        qd = jnp.tanh(gj[3 * D:3 * D + 1] + dfb_ref[0])
        wl = gj[3 * D + 1:3 * D + 1 + M]
        wmax = jnp.max(wl, axis=0, keepdims=True)
        wexp = jnp.exp(wl - wmax)
        wsm = wexp / jnp.sum(wexp, axis=0, keepdims=True)
        w_ref[j] = wsm
        e_ref[j] = e
        a_ref[j] = a
        f2_ref[j] = gj[2 * D:3 * D]
        qd_ref[j] = qd


def _scan_kernel(w_ref, e_ref, a_ref, mv0_ref, rd_ref):
    for mc in range(M // MC):
        def step(t, rows):
            et = e_ref[t]
            at = a_ref[t]
            acc = None
            new_rows = []
            for j in range(MC):
                wt = w_ref[t * M + mc * MC + j]
                row = rows[j]
                z = wt * row
                acc = z if acc is None else acc + z
                new_rows.append(row - wt * (row * et - at))
            if mc == 0:
                rd_ref[t] = acc
            else:
                rd_ref[t] = rd_ref[t] + acc
            return tuple(new_rows)

        init = tuple(mv0_ref[mc * MC + j] for j in range(MC))
        jax.lax.fori_loop(0, L, step, init)


def _post_kernel(rd_ref, f2_ref, qd_ref, fw1_ref, fb_ref, abw_ref, abb_ref,
                 p_ref):
    fw1 = fw1_ref[...]
    abw = abw_ref[...]
    for j in range(LC):
        rd = rd_ref[j]
        f = jnp.tanh(jnp.dot(fw1, rd, preferred_element_type=jnp.float32)
                     + f2_ref[j] + fb_ref[...])
        ab = jnp.tanh(jnp.dot(abw, f, preferred_element_type=jnp.float32)
                      + abb_ref[0])
        p_ref[j] = jax.nn.sigmoid(3.0 * ab - qd_ref[j])


def kernel(q, r, k_emb, v_emb, Mk, Mv0, f_W, f_b, e_W, e_b, a_W, a_b,
           ab_W, ab_b, df_W, df_b):
    f32 = jnp.float32
    qT = q.astype(jnp.int32).T
    rf = r.astype(f32).T.reshape(L, 1, B)
    fW1 = f_W[:, :D]
    fW2 = f_W[:, D:]
    catw = jnp.concatenate(
        [e_W, a_W, fW2, df_W, Mk,
         jnp.zeros((PP - (3 * D + 1 + M), D), f32)], axis=0)
    ev = e_W @ v_emb.T
    av = a_W @ v_emb.T
    const = jnp.stack([
        jnp.broadcast_to((ev[:, 0] + e_b)[:, None], (D, B)),
        jnp.broadcast_to((ev[:, 1] - ev[:, 0])[:, None], (D, B)),
        jnp.broadcast_to((av[:, 0] + a_b)[:, None], (D, B)),
        jnp.broadcast_to((av[:, 1] - av[:, 0])[:, None], (D, B)),
    ])
    dfb = jnp.broadcast_to(df_b.reshape(1, 1), (1, B)).reshape(1, 1, B)
    fb_bc = jnp.broadcast_to(f_b[:, None], (D, B))
    abb = jnp.broadcast_to(ab_b.reshape(1, 1), (1, B)).reshape(1, 1, B)
    mv0b = jnp.broadcast_to(Mv0[:, :, None], (M, D, B))

    w3, e3, a3, f23, qd3 = pl.pallas_call(
        _pre_kernel,
        grid_spec=pltpu.PrefetchScalarGridSpec(
            num_scalar_prefetch=1,
            grid=(L // LCP,),
            in_specs=[
                pl.BlockSpec((LCP, 1, B), lambda l, qs: (l, 0, 0)),
                pl.BlockSpec((NUM_C, D), lambda l, qs: (0, 0)),
                pl.BlockSpec((PP, D), lambda l, qs: (0, 0)),
                pl.BlockSpec((4, D, B), lambda l, qs: (0, 0, 0)),
                pl.BlockSpec((1, 1, B), lambda l, qs: (0, 0, 0)),
            ],
            out_specs=[
                pl.BlockSpec((LCP, M, B), lambda l, qs: (l, 0, 0)),
                pl.BlockSpec((LCP, D, B), lambda l, qs: (l, 0, 0)),
                pl.BlockSpec((LCP, D, B), lambda l, qs: (l, 0, 0)),
                pl.BlockSpec((LCP, D, B), lambda l, qs: (l, 0, 0)),
                pl.BlockSpec((LCP, 1, B), lambda l, qs: (l, 0, 0)),
            ],
            scratch_shapes=[pltpu.VMEM((LCP * B, D), f32)],
        ),
        out_shape=[
            jax.ShapeDtypeStruct((L, M, B), f32),
            jax.ShapeDtypeStruct((L, D, B), f32),
            jax.ShapeDtypeStruct((L, D, B), f32),
            jax.ShapeDtypeStruct((L, D, B), f32),
            jax.ShapeDtypeStruct((L, 1, B), f32),
        ],
        compiler_params=pltpu.CompilerParams(
            dimension_semantics=("parallel",),
            vmem_limit_bytes=48 * 1024 * 1024,
        ),
    )(qT, rf, k_emb, catw, const, dfb)

    wflat = w3.reshape(L * M, 1, B)
    reads = pl.pallas_call(
        _scan_kernel,
        grid=(NDQ,),
        in_specs=[
            pl.BlockSpec((L * M, 1, B), lambda i: (0, 0, 0)),
            pl.BlockSpec((L, DH, B), lambda i: (0, i, 0)),
            pl.BlockSpec((L, DH, B), lambda i: (0, i, 0)),
            pl.BlockSpec((M, DH, B), lambda i: (0, i, 0)),
        ],
        out_specs=pl.BlockSpec((L, DH, B), lambda i: (0, i, 0)),
        out_shape=jax.ShapeDtypeStruct((L, D, B), f32),
        compiler_params=pltpu.CompilerParams(
            dimension_semantics=("parallel",),
            vmem_limit_bytes=48 * 1024 * 1024,
        ),
    )(wflat, e3, a3, mv0b)

    p3 = pl.pallas_call(
        _post_kernel,
        grid=(L // LC,),
        in_specs=[
            pl.BlockSpec((LC, D, B), lambda i: (i, 0, 0)),
            pl.BlockSpec((LC, D, B), lambda i: (i, 0, 0)),
            pl.BlockSpec((LC, 1, B), lambda i: (i, 0, 0)),
            pl.BlockSpec((D, D), lambda i: (0, 0)),
            pl.BlockSpec((D, B), lambda i: (0, 0)),
            pl.BlockSpec((1, D), lambda i: (0, 0)),
            pl.BlockSpec((1, 1, B), lambda i: (0, 0, 0)),
        ],
        out_specs=pl.BlockSpec((LC, 1, B), lambda i: (i, 0, 0)),
        out_shape=jax.ShapeDtypeStruct((L, 1, B), f32),
        compiler_params=pltpu.CompilerParams(
            dimension_semantics=("parallel",),
            vmem_limit_bytes=48 * 1024 * 1024,
        ),
    )(reads, f23, qd3, fW1, fb_bc, ab_W, abb)

    return p3.reshape(L, B).T


# speedup vs baseline: 3.8226x; 1.0233x over previous
import jax
import jax.numpy as jnp
from jax.experimental import pallas as pl
from jax.experimental.pallas import tpu as pltpu

B, L = 128, 200
NUM_C, D, M = 10000, 128, 64
DH = 32
NDQ = D // DH
LC = 25
PP = 456


LCP = 4
MC = 8


def _pre_kernel(q_sref, rf_ref, kemb_ref, catw_ref, const_ref, dfb_ref,
                w_ref, e_ref, a_ref, f2_ref, qd_ref, kbuf):
    l0 = pl.program_id(0) * LCP
    iota8 = jax.lax.broadcasted_iota(jnp.int32, (8, D), 0)
    for j in range(LCP):
        for b in range(B):
            idx = q_sref[l0 + j, b]
            base = pl.multiple_of((idx >> 3) << 3, 8)
            chunk = kemb_ref[pl.ds(base, 8), :]
            row = jnp.sum(jnp.where(iota8 == (idx & 7), chunk, 0.0), axis=0,
                          keepdims=True)
            kbuf[pl.ds(j * B + b, 1), :] = row
    k = kbuf[...]
    g = jax.lax.dot_general(catw_ref[...], k, (((1,), (1,)), ((), ())),
                            preferred_element_type=jnp.float32)
    for j in range(LCP):
        gj = g[:, j * B:(j + 1) * B]
        rf = rf_ref[j]
        e = jax.nn.sigmoid(gj[0:D] + const_ref[0] + rf * const_ref[1])
        a = jnp.tanh(gj[D:2 * D] + const_ref[2] + rf * const_ref[3])
        qd = jnp.tanh(gj[3 * D:3 * D + 1] + dfb_ref[0])
        wl = gj[3 * D + 1:3 * D + 1 + M]
        wmax = jnp.max(wl, axis=0, keepdims=True)
        wexp = jnp.exp(wl - wmax)
        wsm = wexp / jnp.sum(wexp, axis=0, keepdims=True)
        w_ref[j] = wsm
        e_ref[j] = e
        a_ref[j] = a
        f2_ref[j] = gj[2 * D:3 * D]
        qd_ref[j] = qd


def _scan_kernel(w_ref, e_ref, a_ref, mv0_ref, rd_ref):
    for mc in range(M // MC):
        def step(t, rows):
            et = e_ref[t]
            at = a_ref[t]
            acc = None
            new_rows = []
            wts = w_ref[pl.ds(t * M + mc * MC, MC)]
            for j in range(MC):
                wt = wts[j]
                row = rows[j]
                z = wt * row
                acc = z if acc is None else acc + z
                new_rows.append(row - wt * (row * et - at))
            if mc == 0:
                rd_ref[t] = acc
            else:
                rd_ref[t] = rd_ref[t] + acc
            return tuple(new_rows)

        init = tuple(mv0_ref[mc * MC + j] for j in range(MC))
        jax.lax.fori_loop(0, L, step, init)


def _post_kernel(rd_ref, f2_ref, qd_ref, fw1_ref, fb_ref, abw_ref, abb_ref,
                 p_ref):
    fw1 = fw1_ref[...]
    abw = abw_ref[...]
    for j in range(LC):
        rd = rd_ref[j]
        f = jnp.tanh(jnp.dot(fw1, rd, preferred_element_type=jnp.float32)
                     + f2_ref[j] + fb_ref[...])
        ab = jnp.tanh(jnp.dot(abw, f, preferred_element_type=jnp.float32)
                      + abb_ref[0])
        p_ref[j] = jax.nn.sigmoid(3.0 * ab - qd_ref[j])


def kernel(q, r, k_emb, v_emb, Mk, Mv0, f_W, f_b, e_W, e_b, a_W, a_b,
           ab_W, ab_b, df_W, df_b):
    f32 = jnp.float32
    qT = q.astype(jnp.int32).T
    rf = r.astype(f32).T.reshape(L, 1, B)
    fW1 = f_W[:, :D]
    fW2 = f_W[:, D:]
    catw = jnp.concatenate(
        [e_W, a_W, fW2, df_W, Mk,
         jnp.zeros((PP - (3 * D + 1 + M), D), f32)], axis=0)
    ev = e_W @ v_emb.T
    av = a_W @ v_emb.T
    const = jnp.stack([
        jnp.broadcast_to((ev[:, 0] + e_b)[:, None], (D, B)),
        jnp.broadcast_to((ev[:, 1] - ev[:, 0])[:, None], (D, B)),
        jnp.broadcast_to((av[:, 0] + a_b)[:, None], (D, B)),
        jnp.broadcast_to((av[:, 1] - av[:, 0])[:, None], (D, B)),
    ])
    dfb = jnp.broadcast_to(df_b.reshape(1, 1), (1, B)).reshape(1, 1, B)
    fb_bc = jnp.broadcast_to(f_b[:, None], (D, B))
    abb = jnp.broadcast_to(ab_b.reshape(1, 1), (1, B)).reshape(1, 1, B)
    mv0b = jnp.broadcast_to(Mv0[:, :, None], (M, D, B))

    w3, e3, a3, f23, qd3 = pl.pallas_call(
        _pre_kernel,
        grid_spec=pltpu.PrefetchScalarGridSpec(
            num_scalar_prefetch=1,
            grid=(L // LCP,),
            in_specs=[
                pl.BlockSpec((LCP, 1, B), lambda l, qs: (l, 0, 0)),
                pl.BlockSpec((NUM_C, D), lambda l, qs: (0, 0)),
                pl.BlockSpec((PP, D), lambda l, qs: (0, 0)),
                pl.BlockSpec((4, D, B), lambda l, qs: (0, 0, 0)),
                pl.BlockSpec((1, 1, B), lambda l, qs: (0, 0, 0)),
            ],
            out_specs=[
                pl.BlockSpec((LCP, M, B), lambda l, qs: (l, 0, 0)),
                pl.BlockSpec((LCP, D, B), lambda l, qs: (l, 0, 0)),
                pl.BlockSpec((LCP, D, B), lambda l, qs: (l, 0, 0)),
                pl.BlockSpec((LCP, D, B), lambda l, qs: (l, 0, 0)),
                pl.BlockSpec((LCP, 1, B), lambda l, qs: (l, 0, 0)),
            ],
            scratch_shapes=[pltpu.VMEM((LCP * B, D), f32)],
        ),
        out_shape=[
            jax.ShapeDtypeStruct((L, M, B), f32),
            jax.ShapeDtypeStruct((L, D, B), f32),
            jax.ShapeDtypeStruct((L, D, B), f32),
            jax.ShapeDtypeStruct((L, D, B), f32),
            jax.ShapeDtypeStruct((L, 1, B), f32),
        ],
        compiler_params=pltpu.CompilerParams(
            dimension_semantics=("parallel",),
            vmem_limit_bytes=48 * 1024 * 1024,
        ),
    )(qT, rf, k_emb, catw, const, dfb)

    wflat = w3.reshape(L * M, 1, B)
    reads = pl.pallas_call(
        _scan_kernel,
        grid=(NDQ,),
        in_specs=[
            pl.BlockSpec((L * M, 1, B), lambda i: (0, 0, 0)),
            pl.BlockSpec((L, DH, B), lambda i: (0, i, 0)),
            pl.BlockSpec((L, DH, B), lambda i: (0, i, 0)),
            pl.BlockSpec((M, DH, B), lambda i: (0, i, 0)),
        ],
        out_specs=pl.BlockSpec((L, DH, B), lambda i: (0, i, 0)),
        out_shape=jax.ShapeDtypeStruct((L, D, B), f32),
        compiler_params=pltpu.CompilerParams(
            dimension_semantics=("parallel",),
            vmem_limit_bytes=48 * 1024 * 1024,
        ),
    )(wflat, e3, a3, mv0b)

    p3 = pl.pallas_call(
        _post_kernel,
        grid=(L // LC,),
        in_specs=[
            pl.BlockSpec((LC, D, B), lambda i: (i, 0, 0)),
            pl.BlockSpec((LC, D, B), lambda i: (i, 0, 0)),
            pl.BlockSpec((LC, 1, B), lambda i: (i, 0, 0)),
            pl.BlockSpec((D, D), lambda i: (0, 0)),
            pl.BlockSpec((D, B), lambda i: (0, 0)),
            pl.BlockSpec((1, D), lambda i: (0, 0)),
            pl.BlockSpec((1, 1, B), lambda i: (0, 0, 0)),
        ],
        out_specs=pl.BlockSpec((LC, 1, B), lambda i: (i, 0, 0)),
        out_shape=jax.ShapeDtypeStruct((L, 1, B), f32),
        compiler_params=pltpu.CompilerParams(
            dimension_semantics=("parallel",),
            vmem_limit_bytes=48 * 1024 * 1024,
        ),
    )(reads, f23, qd3, fW1, fb_bc, ab_W, abb)

    return p3.reshape(L, B).T
